# baseline (speedup 1.0000x reference)
import functools

import jax
import jax.numpy as jnp
from jax import lax
from jax.experimental import pallas as pl
from jax.experimental.pallas import tpu as pltpu
from jax.experimental.pallas import tpu_sc as plsc

NUM_ENT = 100000
NUM_REL = 1000
ENT_DIM = 128
REL_DIM = 256
BATCH = 16384

NC = 2
NS = 16
NW = NC * NS
BPW = BATCH // NW
CW = 128
NCHUNK = BPW // CW
NBUF = 7
NTASK = 4 * NCHUNK
DLAG = 2


def _body(h_idx, t_idx, r_idx, ent, rel2, out_h, out_t, out_r,
          idx_h, idx_t, idx_r, idx_ra, idx_rb, bufs, gsem, wsem):
    wid = lax.axis_index("s") * NC + lax.axis_index("c")
    blk = wid * NCHUNK
    base = wid * BPW

    tasks = []
    for j in range(NCHUNK):
        tasks.append((ent, idx_h.at[j], out_h, base + j * CW, 0))
    for j in range(NCHUNK):
        tasks.append((ent, idx_t.at[j], out_t, base + j * CW, 0))
    for j in range(NCHUNK):
        tasks.append((rel2, idx_ra.at[j], out_r, base + j * CW, 0))
    for j in range(NCHUNK):
        tasks.append((rel2, idx_rb.at[j], out_r, base + j * CW, CW))

    def gather(k, b):
        tbl, idx, _, _, _ = tasks[k]
        return pltpu.make_async_copy(tbl.at[idx], bufs.at[b], gsem.at[b])

    def write(k, b):
        _, _, out, off, col = tasks[k]
        dst = out.at[pl.ds(off, CW), pl.ds(col, CW)]
        return pltpu.make_async_copy(bufs.at[b], dst, wsem.at[b])

    pltpu.sync_copy(h_idx.at[pl.ds(blk, NCHUNK)], idx_h)
    for k in range(NCHUNK):
        gather(k, k).start()
    pltpu.sync_copy(t_idx.at[pl.ds(blk, NCHUNK)], idx_t)
    for k in range(NCHUNK, NBUF):
        gather(k, k).start()
    pltpu.sync_copy(r_idx.at[pl.ds(blk, NCHUNK)], idx_r)
    for j in range(NCHUNK):
        for i in range(CW // 16):
            v = idx_r[j, pl.ds(i * 16, 16)]
            idx_ra[j, pl.ds(i * 16, 16)] = v * 2
            idx_rb[j, pl.ds(i * 16, 16)] = v * 2 + 1

    for k in range(NTASK):
        b = k % NBUF
        rk = k - DLAG + NBUF
        if k >= DLAG and NBUF <= rk < NTASK:
            pb = (k - DLAG) % NBUF
            write(k - DLAG, pb).wait()
            gather(rk, pb).start()
        gather(k, b).wait()
        write(k, b).start()
    waited = set()
    for k in range(NTASK):
        rk = k - DLAG + NBUF
        if k >= DLAG and NBUF <= rk < NTASK:
            waited.add(k - DLAG)
    for k in range(NTASK):
        if k not in waited:
            write(k, k % NBUF).wait()


@jax.jit
def _gather3(h_idx, t_idx, r_idx, ent_table, rel2):
    mesh = plsc.VectorSubcoreMesh(core_axis_name="c", subcore_axis_name="s")
    k = pl.kernel(
        _body,
        out_type=(
            jax.ShapeDtypeStruct((BATCH, ENT_DIM), jnp.float32),
            jax.ShapeDtypeStruct((BATCH, ENT_DIM), jnp.float32),
            jax.ShapeDtypeStruct((BATCH, REL_DIM), jnp.float32),
        ),
        mesh=mesh,
        scratch_types=[
            pltpu.VMEM((NCHUNK, CW), jnp.int32),
            pltpu.VMEM((NCHUNK, CW), jnp.int32),
            pltpu.VMEM((NCHUNK, CW), jnp.int32),
            pltpu.VMEM((NCHUNK, CW), jnp.int32),
            pltpu.VMEM((NCHUNK, CW), jnp.int32),
            pltpu.VMEM((NBUF, CW, ENT_DIM), jnp.float32),
            pltpu.SemaphoreType.DMA((NBUF,)),
            pltpu.SemaphoreType.DMA((NBUF,)),
        ],
    )
    return k(h_idx, t_idx, r_idx, ent_table, rel2)


def kernel(batch_h, batch_t, batch_r, mode, ent_table, rel_table):
    del mode
    h2 = batch_h.reshape(BATCH // CW, CW)
    t2 = batch_t.reshape(BATCH // CW, CW)
    r2 = batch_r.reshape(BATCH // CW, CW)
    rel2 = rel_table.reshape(NUM_REL * 2, ENT_DIM)
    return _gather3(h2, t2, r2, ent_table, rel2)


# speedup vs baseline: 1.0259x; 1.0259x over previous
import functools

import jax
import jax.numpy as jnp
from jax import lax
from jax.experimental import pallas as pl
from jax.experimental.pallas import tpu as pltpu
from jax.experimental.pallas import tpu_sc as plsc

NUM_ENT = 100000
NUM_REL = 1000
ENT_DIM = 128
REL_DIM = 256
BATCH = 16384

NC = 2
NS = 16
NW = NC * NS
BPW = BATCH // NW
CWE = 128
CWR = 64
NBE = 4
NBR = 3


def _body(h_idx, t_idx, r_idx, ent, rel, out_h, out_t, out_r,
          idx_h, idx_t, idx_r, bufs_e, bufs_r, gsem_e, wsem_e, gsem_r, wsem_r):
    wid = lax.axis_index("s") * NC + lax.axis_index("c")
    base = wid * BPW
    pltpu.sync_copy(h_idx.at[pl.ds(base, BPW)], idx_h)
    pltpu.sync_copy(t_idx.at[pl.ds(base, BPW)], idx_t)
    pltpu.sync_copy(r_idx.at[pl.ds(base, BPW)], idx_r)

    etasks = []
    for j in range(BPW // CWE):
        etasks.append((idx_h.at[pl.ds(j * CWE, CWE)], out_h, base + j * CWE))
        etasks.append((idx_t.at[pl.ds(j * CWE, CWE)], out_t, base + j * CWE))
    rtasks = []
    for j in range(BPW // CWR):
        rtasks.append((idx_r.at[pl.ds(j * CWR, CWR)], out_r, base + j * CWR))

    def egather(i):
        idx, _, _ = etasks[i]
        b = i % NBE
        return pltpu.make_async_copy(ent.at[idx], bufs_e.at[b], gsem_e.at[b])

    def ewrite(i):
        _, out, off = etasks[i]
        b = i % NBE
        return pltpu.make_async_copy(
            bufs_e.at[b], out.at[pl.ds(off, CWE)], wsem_e.at[b])

    def rgather(i):
        idx, _, _ = rtasks[i]
        b = i % NBR
        return pltpu.make_async_copy(rel.at[idx], bufs_r.at[b], gsem_r.at[b])

    def rwrite(i):
        _, out, off = rtasks[i]
        b = i % NBR
        return pltpu.make_async_copy(
            bufs_r.at[b], out.at[pl.ds(off, CWR)], wsem_r.at[b])

    NE, NR = len(etasks), len(rtasks)
    order = []
    for j in range(4):
        order += [("e", 2 * j), ("e", 2 * j + 1), ("r", 2 * j), ("r", 2 * j + 1)]

    for i in range(NBE):
        egather(i).start()
    for i in range(NBR):
        rgather(i).start()

    ewaited, rwaited = set(), set()
    for ring, i in order:
        if ring == "e":
            nk = i + NBE - 1
            if i >= 1 and nk < NE:
                ewrite(i - 1).wait()
                ewaited.add(i - 1)
                egather(nk).start()
            egather(i).wait()
            ewrite(i).start()
        else:
            nk = i + NBR - 1
            if i >= 1 and nk < NR:
                rwrite(i - 1).wait()
                rwaited.add(i - 1)
                rgather(nk).start()
            rgather(i).wait()
            rwrite(i).start()
    for i in range(NE):
        if i not in ewaited:
            ewrite(i).wait()
    for i in range(NR):
        if i not in rwaited:
            rwrite(i).wait()


@jax.jit
def _gather3(h_idx, t_idx, r_idx, ent_table, rel_table):
    mesh = plsc.VectorSubcoreMesh(core_axis_name="c", subcore_axis_name="s")
    k = pl.kernel(
        _body,
        out_type=(
            jax.ShapeDtypeStruct((BATCH, ENT_DIM), jnp.float32),
            jax.ShapeDtypeStruct((BATCH, ENT_DIM), jnp.float32),
            jax.ShapeDtypeStruct((BATCH, REL_DIM), jnp.float32),
        ),
        mesh=mesh,
        scratch_types=[
            pltpu.VMEM((BPW,), jnp.int32),
            pltpu.VMEM((BPW,), jnp.int32),
            pltpu.VMEM((BPW,), jnp.int32),
            pltpu.VMEM((NBE, CWE, ENT_DIM), jnp.float32),
            pltpu.VMEM((NBR, CWR, REL_DIM), jnp.float32),
            pltpu.SemaphoreType.DMA((NBE,)),
            pltpu.SemaphoreType.DMA((NBE,)),
            pltpu.SemaphoreType.DMA((NBR,)),
            pltpu.SemaphoreType.DMA((NBR,)),
        ],
    )
    return k(h_idx, t_idx, r_idx, ent_table, rel_table)


def kernel(batch_h, batch_t, batch_r, mode, ent_table, rel_table):
    del mode
    return _gather3(batch_h, batch_t, batch_r, ent_table, rel_table)
